# baseline (speedup 1.0000x reference)
import jax
import jax.numpy as jnp
import numpy as np
from jax import lax
from jax.experimental import pallas as pl
from jax.experimental.pallas import tpu as pltpu
from jax.experimental.pallas import tpu_sc as plsc

_D = 128
_K = 10
_B = 4096
_M = _B * (_K + 1)
_NW = 32
_PER_TILE = _M // _NW
_C = 32
_NCHUNK = _PER_TILE // _C
_NIDX = _C * (_K + 1)
_JROWS = 4
_JW = _NIDX // _JROWS


def _build_dmap():
    p = np.arange(_M)
    pn = p - _B
    r = np.where(p < _B, p, (pn % _K + 1) * _B + pn // _K)
    return r.reshape(_NW, _NCHUNK, _C).astype(np.int32)


_DMAP = _build_dmap()


def _sc_gather_mean(raw_hbm, cidx_hbm, dmap_hbm,
                    self_out, agg_out,
                    cidx_v, dmap_v, rows0, rows1, ag0, ag1,
                    sg0, sg1, sws0, sws1, swa0, swa1):
    cid = lax.axis_index("c")
    sid = lax.axis_index("s")
    wid = sid * 2 + cid

    rows = (rows0, rows1)
    aggs = (ag0, ag1)
    sem_g = (sg0, sg1)
    sem_ws = (sws0, sws1)
    sem_wa = (swa0, swa1)

    pltpu.sync_copy(cidx_hbm.at[wid], cidx_v)
    pltpu.sync_copy(dmap_hbm.at[wid], dmap_v)

    def g_issue(c, rows_ref, sem):
        for j in range(_JROWS):
            pltpu.async_copy(raw_hbm.at[cidx_v.at[c * _JROWS + j]],
                             rows_ref.at[pl.ds(j * _JW, _JW)], sem)

    def g_wait(c, rows_ref, sem):
        for j in range(_JROWS):
            pltpu.make_async_copy(raw_hbm.at[cidx_v.at[c * _JROWS + j]],
                                  rows_ref.at[pl.ds(j * _JW, _JW)], sem).wait()

    def ws_issue(c, rows_ref, sem):
        pltpu.async_copy(rows_ref.at[pl.ds(_C * _K, _C)],
                         self_out.at[dmap_v.at[c]], sem)

    def wa_issue(c, agg_ref, sem):
        pltpu.async_copy(agg_ref, agg_out.at[dmap_v.at[c]], sem)

    def ws_wait(rows_ref, sem):
        pltpu.make_async_copy(rows_ref.at[pl.ds(_C * _K, _C)],
                              self_out.at[dmap_v.at[0]], sem).wait()

    def wa_wait(agg_ref, sem):
        pltpu.make_async_copy(agg_ref, agg_out.at[dmap_v.at[0]], sem).wait()

    def compute(rows_ref, agg_ref):
        def node_body(n, carry):
            base = n * _K
            for d in range(_D // 16):
                sl = pl.ds(d * 16, 16)
                acc = rows_ref[base, sl]
                for k in range(1, _K):
                    acc = acc + rows_ref[base + k, sl]
                agg_ref[n, sl] = acc
            return carry

        lax.fori_loop(0, _C, node_body, 0, unroll=2)

    def step(c, u, do_ws_wait, do_wa_wait, issue_next, when_pred=None):
        ru = u % 2
        ru1 = 1 - ru
        g_wait(c, rows[ru], sem_g[ru])
        ws_issue(c, rows[ru], sem_ws[ru])
        if issue_next:
            def _issue():
                if do_ws_wait:
                    ws_wait(rows[ru1], sem_ws[ru1])
                g_issue(c + 1, rows[ru1], sem_g[ru1])
            if when_pred is None:
                _issue()
            else:
                pl.when(when_pred)(_issue)
        if do_wa_wait:
            wa_wait(aggs[ru], sem_wa[ru])
        compute(rows[ru], aggs[ru])
        wa_issue(c, aggs[ru], sem_wa[ru])

    g_issue(0, rows0, sg0)
    step(0, 0, do_ws_wait=False, do_wa_wait=False, issue_next=True)
    step(1, 1, do_ws_wait=True, do_wa_wait=False, issue_next=True)

    def loop_body(t, carry):
        c0 = t * 2
        step(c0, 0, True, True, True)
        step(c0 + 1, 1, True, True, True, when_pred=t < (_NCHUNK // 2 - 1))
        return carry

    lax.fori_loop(1, _NCHUNK // 2, loop_body, 0)

    ws_wait(rows0, sws0)
    ws_wait(rows1, sws1)
    wa_wait(ag0, swa0)
    wa_wait(ag1, swa1)


def _tc_sage(self_ref, agg_ref, w1a, w1b, w2a, w2b, out_ref):
    h1s = jnp.maximum(
        jnp.dot(self_ref[0], w1a[...], preferred_element_type=jnp.float32)
        + jnp.dot(agg_ref[0], w1b[...], preferred_element_type=jnp.float32),
        0.0,
    )
    acc = jnp.zeros_like(h1s)
    for k in range(_K):
        hk = jnp.maximum(
            jnp.dot(self_ref[k + 1], w1a[...], preferred_element_type=jnp.float32)
            + jnp.dot(agg_ref[k + 1], w1b[...], preferred_element_type=jnp.float32),
            0.0,
        )
        acc = acc + hk
    out_ref[...] = jnp.maximum(
        jnp.dot(h1s, w2a[...], preferred_element_type=jnp.float32)
        + jnp.dot(acc, w2b[...], preferred_element_type=jnp.float32),
        0.0,
    )


def kernel(nodes_batch, raw_features, l1_nodes, neigh_l1, W1, W2):
    del nodes_batch
    cidx = jnp.concatenate(
        [neigh_l1.astype(jnp.int32).reshape(_NW * _NCHUNK, _C * _K),
         l1_nodes.astype(jnp.int32).reshape(_NW * _NCHUNK, _C)],
        axis=1,
    ).reshape(_NW, _NCHUNK * _JROWS, _JW)

    sc_fn = pl.kernel(
        _sc_gather_mean,
        out_type=(
            jax.ShapeDtypeStruct(((_K + 1) * _B, _D), jnp.float32),
            jax.ShapeDtypeStruct(((_K + 1) * _B, _D), jnp.float32),
        ),
        mesh=plsc.VectorSubcoreMesh(core_axis_name="c", subcore_axis_name="s"),
        scratch_types=[
            pltpu.VMEM((_NCHUNK * _JROWS, _JW), jnp.int32),
            pltpu.VMEM((_NCHUNK, _C), jnp.int32),
            pltpu.VMEM((_NIDX, _D), jnp.float32),
            pltpu.VMEM((_NIDX, _D), jnp.float32),
            pltpu.VMEM((_C, _D), jnp.float32),
            pltpu.VMEM((_C, _D), jnp.float32),
            pltpu.SemaphoreType.DMA,
            pltpu.SemaphoreType.DMA,
            pltpu.SemaphoreType.DMA,
            pltpu.SemaphoreType.DMA,
            pltpu.SemaphoreType.DMA,
            pltpu.SemaphoreType.DMA,
        ],
    )
    selfp, aggp = sc_fn(raw_features, cidx, jnp.asarray(_DMAP))

    w1a = W1[:, :_D].T
    w1b = W1[:, _D:].T * jnp.float32(1.0 / _K)
    w2a = W2[:, :_D].T
    w2b = W2[:, _D:].T * jnp.float32(1.0 / _K)
    selfp = selfp.reshape(_K + 1, _B, _D)
    aggp = aggp.reshape(_K + 1, _B, _D)

    S = 512
    grid = _B // S
    h2 = pl.pallas_call(
        _tc_sage,
        grid=(grid,),
        in_specs=[
            pl.BlockSpec((_K + 1, S, _D), lambda g: (0, g, 0)),
            pl.BlockSpec((_K + 1, S, _D), lambda g: (0, g, 0)),
            pl.BlockSpec((_D, _D), lambda g: (0, 0)),
            pl.BlockSpec((_D, _D), lambda g: (0, 0)),
            pl.BlockSpec((_D, _D), lambda g: (0, 0)),
            pl.BlockSpec((_D, _D), lambda g: (0, 0)),
        ],
        out_specs=pl.BlockSpec((S, _D), lambda g: (g, 0)),
        out_shape=jax.ShapeDtypeStruct((_B, _D), jnp.float32),
    )(selfp, aggp, w1a, w1b, w2a, w2b)
    return h2


# speedup vs baseline: 1.0060x; 1.0060x over previous
import jax
import jax.numpy as jnp
import numpy as np
from jax import lax
from jax.experimental import pallas as pl
from jax.experimental.pallas import tpu as pltpu
from jax.experimental.pallas import tpu_sc as plsc

_D = 128
_K = 10
_B = 4096
_M = _B * (_K + 1)
_NW = 32
_PER_TILE = _M // _NW
_C = 32
_NCHUNK = _PER_TILE // _C
_NIDX = _C * (_K + 1)
_JROWS = 4
_JW = _NIDX // _JROWS


def _build_dmap():
    p = np.arange(_M)
    pn = p - _B
    r = np.where(p < _B, p, (pn % _K + 1) * _B + pn // _K)
    return r.reshape(_NW, _NCHUNK, _C).astype(np.int32)


_DMAP = _build_dmap()


def _sc_gather_mean(raw_hbm, cidx_hbm, dmap_hbm,
                    self_out, agg_out,
                    cidx_v, dmap_v, rows0, rows1, ag0, ag1,
                    sg0, sg1, sws0, sws1, swa0, swa1):
    cid = lax.axis_index("c")
    sid = lax.axis_index("s")
    wid = sid * 2 + cid

    rows = (rows0, rows1)
    aggs = (ag0, ag1)
    sem_g = (sg0, sg1)
    sem_ws = (sws0, sws1)
    sem_wa = (swa0, swa1)

    pltpu.sync_copy(cidx_hbm.at[wid], cidx_v)
    pltpu.sync_copy(dmap_hbm.at[wid], dmap_v)

    def g_issue(c, rows_ref, sem):
        for j in range(_JROWS):
            pltpu.async_copy(raw_hbm.at[cidx_v.at[c * _JROWS + j]],
                             rows_ref.at[pl.ds(j * _JW, _JW)], sem)

    def g_wait(c, rows_ref, sem):
        for j in range(_JROWS):
            pltpu.make_async_copy(raw_hbm.at[cidx_v.at[c * _JROWS + j]],
                                  rows_ref.at[pl.ds(j * _JW, _JW)], sem).wait()

    def ws_issue(c, rows_ref, sem):
        pltpu.async_copy(rows_ref.at[pl.ds(_C * _K, _C)],
                         self_out.at[dmap_v.at[c]], sem)

    def wa_issue(c, agg_ref, sem):
        pltpu.async_copy(agg_ref, agg_out.at[dmap_v.at[c]], sem)

    def ws_wait(rows_ref, sem):
        pltpu.make_async_copy(rows_ref.at[pl.ds(_C * _K, _C)],
                              self_out.at[dmap_v.at[0]], sem).wait()

    def wa_wait(agg_ref, sem):
        pltpu.make_async_copy(agg_ref, agg_out.at[dmap_v.at[0]], sem).wait()

    def compute(rows_ref, agg_ref):
        def node_body(n, carry):
            base = n * _K
            for d in range(_D // 16):
                sl = pl.ds(d * 16, 16)
                acc = rows_ref[base, sl]
                for k in range(1, _K):
                    acc = acc + rows_ref[base + k, sl]
                agg_ref[n, sl] = acc
            return carry

        lax.fori_loop(0, _C, node_body, 0, unroll=False)

    def step(c, u, do_ws_wait, do_wa_wait, issue_next, when_pred=None):
        ru = u % 2
        ru1 = 1 - ru
        g_wait(c, rows[ru], sem_g[ru])
        ws_issue(c, rows[ru], sem_ws[ru])
        if issue_next:
            def _issue():
                if do_ws_wait:
                    ws_wait(rows[ru1], sem_ws[ru1])
                g_issue(c + 1, rows[ru1], sem_g[ru1])
            if when_pred is None:
                _issue()
            else:
                pl.when(when_pred)(_issue)
        if do_wa_wait:
            wa_wait(aggs[ru], sem_wa[ru])
        compute(rows[ru], aggs[ru])
        wa_issue(c, aggs[ru], sem_wa[ru])

    g_issue(0, rows0, sg0)
    step(0, 0, do_ws_wait=False, do_wa_wait=False, issue_next=True)
    step(1, 1, do_ws_wait=True, do_wa_wait=False, issue_next=True)

    def loop_body(t, carry):
        c0 = t * 2
        step(c0, 0, True, True, True)
        step(c0 + 1, 1, True, True, True, when_pred=t < (_NCHUNK // 2 - 1))
        return carry

    lax.fori_loop(1, _NCHUNK // 2, loop_body, 0)

    ws_wait(rows0, sws0)
    ws_wait(rows1, sws1)
    wa_wait(ag0, swa0)
    wa_wait(ag1, swa1)


def _tc_sage(self_ref, agg_ref, w1a, w1b, w2a, w2b, out_ref):
    h1s = jnp.maximum(
        jnp.dot(self_ref[0], w1a[...], preferred_element_type=jnp.float32)
        + jnp.dot(agg_ref[0], w1b[...], preferred_element_type=jnp.float32),
        0.0,
    )
    acc = jnp.zeros_like(h1s)
    for k in range(_K):
        hk = jnp.maximum(
            jnp.dot(self_ref[k + 1], w1a[...], preferred_element_type=jnp.float32)
            + jnp.dot(agg_ref[k + 1], w1b[...], preferred_element_type=jnp.float32),
            0.0,
        )
        acc = acc + hk
    out_ref[...] = jnp.maximum(
        jnp.dot(h1s, w2a[...], preferred_element_type=jnp.float32)
        + jnp.dot(acc, w2b[...], preferred_element_type=jnp.float32),
        0.0,
    )


def kernel(nodes_batch, raw_features, l1_nodes, neigh_l1, W1, W2):
    del nodes_batch
    cidx = jnp.concatenate(
        [neigh_l1.astype(jnp.int32).reshape(_NW * _NCHUNK, _C * _K),
         l1_nodes.astype(jnp.int32).reshape(_NW * _NCHUNK, _C)],
        axis=1,
    ).reshape(_NW, _NCHUNK * _JROWS, _JW)

    sc_fn = pl.kernel(
        _sc_gather_mean,
        out_type=(
            jax.ShapeDtypeStruct(((_K + 1) * _B, _D), jnp.float32),
            jax.ShapeDtypeStruct(((_K + 1) * _B, _D), jnp.float32),
        ),
        mesh=plsc.VectorSubcoreMesh(core_axis_name="c", subcore_axis_name="s"),
        scratch_types=[
            pltpu.VMEM((_NCHUNK * _JROWS, _JW), jnp.int32),
            pltpu.VMEM((_NCHUNK, _C), jnp.int32),
            pltpu.VMEM((_NIDX, _D), jnp.float32),
            pltpu.VMEM((_NIDX, _D), jnp.float32),
            pltpu.VMEM((_C, _D), jnp.float32),
            pltpu.VMEM((_C, _D), jnp.float32),
            pltpu.SemaphoreType.DMA,
            pltpu.SemaphoreType.DMA,
            pltpu.SemaphoreType.DMA,
            pltpu.SemaphoreType.DMA,
            pltpu.SemaphoreType.DMA,
            pltpu.SemaphoreType.DMA,
        ],
    )
    selfp, aggp = sc_fn(raw_features, cidx, jnp.asarray(_DMAP))

    w1a = W1[:, :_D].T
    w1b = W1[:, _D:].T * jnp.float32(1.0 / _K)
    w2a = W2[:, :_D].T
    w2b = W2[:, _D:].T * jnp.float32(1.0 / _K)
    selfp = selfp.reshape(_K + 1, _B, _D)
    aggp = aggp.reshape(_K + 1, _B, _D)

    S = 512
    grid = _B // S
    h2 = pl.pallas_call(
        _tc_sage,
        grid=(grid,),
        in_specs=[
            pl.BlockSpec((_K + 1, S, _D), lambda g: (0, g, 0)),
            pl.BlockSpec((_K + 1, S, _D), lambda g: (0, g, 0)),
            pl.BlockSpec((_D, _D), lambda g: (0, 0)),
            pl.BlockSpec((_D, _D), lambda g: (0, 0)),
            pl.BlockSpec((_D, _D), lambda g: (0, 0)),
            pl.BlockSpec((_D, _D), lambda g: (0, 0)),
        ],
        out_specs=pl.BlockSpec((S, _D), lambda g: (g, 0)),
        out_shape=jax.ShapeDtypeStruct((_B, _D), jnp.float32),
    )(selfp, aggp, w1a, w1b, w2a, w2b)
    return h2


# speedup vs baseline: 1.0185x; 1.0124x over previous
import jax
import jax.numpy as jnp
import numpy as np
from jax import lax
from jax.experimental import pallas as pl
from jax.experimental.pallas import tpu as pltpu
from jax.experimental.pallas import tpu_sc as plsc

_D = 128
_K = 10
_B = 4096
_M = _B * (_K + 1)
_NW = 32
_PER_TILE = _M // _NW
_C = 16
_NCHUNK = _PER_TILE // _C
_NIDX = _C * (_K + 1)
_JROWS = 2
_JW = (_C * _K) // _JROWS


def _build_dmap():
    p = np.arange(_M)
    pn = p - _B
    r = np.where(p < _B, p, (pn % _K + 1) * _B + pn // _K)
    return r.reshape(_NW, _NCHUNK, _C).astype(np.int32)


_DMAP = _build_dmap()


def _sc_gather_mean(raw_hbm, nidx_hbm, sidx_hbm, dmap_hbm,
                    self_out, agg_out,
                    nidx_v, sidx_v, dmap_v, rows0, rows1, ag0, ag1,
                    sg0, sg1, sws0, sws1, swa0, swa1):
    cid = lax.axis_index("c")
    sid = lax.axis_index("s")
    wid = sid * 2 + cid

    rows = (rows0, rows1)
    aggs = (ag0, ag1)
    sem_g = (sg0, sg1)
    sem_ws = (sws0, sws1)
    sem_wa = (swa0, swa1)

    pltpu.sync_copy(nidx_hbm.at[wid], nidx_v)
    pltpu.sync_copy(sidx_hbm.at[wid], sidx_v)
    pltpu.sync_copy(dmap_hbm.at[wid], dmap_v)

    def g_issue(c, rows_ref, sem):
        for j in range(_JROWS):
            pltpu.async_copy(raw_hbm.at[nidx_v.at[c * _JROWS + j]],
                             rows_ref.at[pl.ds(j * _JW, _JW)], sem)
        pltpu.async_copy(raw_hbm.at[sidx_v.at[c]],
                         rows_ref.at[pl.ds(_C * _K, _C)], sem)

    def g_wait(c, rows_ref, sem):
        for j in range(_JROWS):
            pltpu.make_async_copy(raw_hbm.at[nidx_v.at[c * _JROWS + j]],
                                  rows_ref.at[pl.ds(j * _JW, _JW)], sem).wait()
        pltpu.make_async_copy(raw_hbm.at[sidx_v.at[c]],
                              rows_ref.at[pl.ds(_C * _K, _C)], sem).wait()

    def ws_issue(c, rows_ref, sem):
        pltpu.async_copy(rows_ref.at[pl.ds(_C * _K, _C)],
                         self_out.at[dmap_v.at[c]], sem)

    def wa_issue(c, agg_ref, sem):
        pltpu.async_copy(agg_ref, agg_out.at[dmap_v.at[c]], sem)

    def ws_wait(rows_ref, sem):
        pltpu.make_async_copy(rows_ref.at[pl.ds(_C * _K, _C)],
                              self_out.at[dmap_v.at[0]], sem).wait()

    def wa_wait(agg_ref, sem):
        pltpu.make_async_copy(agg_ref, agg_out.at[dmap_v.at[0]], sem).wait()

    def compute(rows_ref, agg_ref):
        def node_body(n, carry):
            base = n * _K
            for d in range(_D // 16):
                sl = pl.ds(d * 16, 16)
                acc = rows_ref[base, sl]
                for k in range(1, _K):
                    acc = acc + rows_ref[base + k, sl]
                agg_ref[n, sl] = acc
            return carry

        lax.fori_loop(0, _C, node_body, 0, unroll=False)

    def step(c, u, do_ws_wait, do_wa_wait, issue_next, when_pred=None):
        ru = u % 2
        ru1 = 1 - ru
        g_wait(c, rows[ru], sem_g[ru])
        ws_issue(c, rows[ru], sem_ws[ru])
        if issue_next:
            def _issue():
                if do_ws_wait:
                    ws_wait(rows[ru1], sem_ws[ru1])
                g_issue(c + 1, rows[ru1], sem_g[ru1])
            if when_pred is None:
                _issue()
            else:
                pl.when(when_pred)(_issue)
        if do_wa_wait:
            wa_wait(aggs[ru], sem_wa[ru])
        compute(rows[ru], aggs[ru])
        wa_issue(c, aggs[ru], sem_wa[ru])

    g_issue(0, rows0, sg0)
    step(0, 0, do_ws_wait=False, do_wa_wait=False, issue_next=True)
    step(1, 1, do_ws_wait=True, do_wa_wait=False, issue_next=True)

    def loop_body(t, carry):
        c0 = t * 2
        step(c0, 0, True, True, True)
        step(c0 + 1, 1, True, True, True, when_pred=t < (_NCHUNK // 2 - 1))
        return carry

    lax.fori_loop(1, _NCHUNK // 2, loop_body, 0)

    ws_wait(rows0, sws0)
    ws_wait(rows1, sws1)
    wa_wait(ag0, swa0)
    wa_wait(ag1, swa1)


def _tc_sage(self_ref, agg_ref, w1a, w1b, w2a, w2b, out_ref):
    h1s = jnp.maximum(
        jnp.dot(self_ref[0], w1a[...], preferred_element_type=jnp.float32)
        + jnp.dot(agg_ref[0], w1b[...], preferred_element_type=jnp.float32),
        0.0,
    )
    acc = jnp.zeros_like(h1s)
    for k in range(_K):
        hk = jnp.maximum(
            jnp.dot(self_ref[k + 1], w1a[...], preferred_element_type=jnp.float32)
            + jnp.dot(agg_ref[k + 1], w1b[...], preferred_element_type=jnp.float32),
            0.0,
        )
        acc = acc + hk
    out_ref[...] = jnp.maximum(
        jnp.dot(h1s, w2a[...], preferred_element_type=jnp.float32)
        + jnp.dot(acc, w2b[...], preferred_element_type=jnp.float32),
        0.0,
    )


def kernel(nodes_batch, raw_features, l1_nodes, neigh_l1, W1, W2):
    del nodes_batch
    nidx = neigh_l1.astype(jnp.int32).reshape(_NW, _NCHUNK * _JROWS, _JW)
    sidx = l1_nodes.astype(jnp.int32).reshape(_NW, _NCHUNK, _C)

    sc_fn = pl.kernel(
        _sc_gather_mean,
        out_type=(
            jax.ShapeDtypeStruct(((_K + 1) * _B, _D), jnp.float32),
            jax.ShapeDtypeStruct(((_K + 1) * _B, _D), jnp.float32),
        ),
        mesh=plsc.VectorSubcoreMesh(core_axis_name="c", subcore_axis_name="s"),
        scratch_types=[
            pltpu.VMEM((_NCHUNK * _JROWS, _JW), jnp.int32),
            pltpu.VMEM((_NCHUNK, _C), jnp.int32),
            pltpu.VMEM((_NCHUNK, _C), jnp.int32),
            pltpu.VMEM((_NIDX, _D), jnp.float32),
            pltpu.VMEM((_NIDX, _D), jnp.float32),
            pltpu.VMEM((_C, _D), jnp.float32),
            pltpu.VMEM((_C, _D), jnp.float32),
            pltpu.SemaphoreType.DMA,
            pltpu.SemaphoreType.DMA,
            pltpu.SemaphoreType.DMA,
            pltpu.SemaphoreType.DMA,
            pltpu.SemaphoreType.DMA,
            pltpu.SemaphoreType.DMA,
        ],
    )
    selfp, aggp = sc_fn(raw_features, nidx, sidx, jnp.asarray(_DMAP))

    w1a = W1[:, :_D].T
    w1b = W1[:, _D:].T * jnp.float32(1.0 / _K)
    w2a = W2[:, :_D].T
    w2b = W2[:, _D:].T * jnp.float32(1.0 / _K)
    selfp = selfp.reshape(_K + 1, _B, _D)
    aggp = aggp.reshape(_K + 1, _B, _D)

    S = 512
    grid = _B // S
    h2 = pl.pallas_call(
        _tc_sage,
        grid=(grid,),
        in_specs=[
            pl.BlockSpec((_K + 1, S, _D), lambda g: (0, g, 0)),
            pl.BlockSpec((_K + 1, S, _D), lambda g: (0, g, 0)),
            pl.BlockSpec((_D, _D), lambda g: (0, 0)),
            pl.BlockSpec((_D, _D), lambda g: (0, 0)),
            pl.BlockSpec((_D, _D), lambda g: (0, 0)),
            pl.BlockSpec((_D, _D), lambda g: (0, 0)),
        ],
        out_specs=pl.BlockSpec((S, _D), lambda g: (g, 0)),
        out_shape=jax.ShapeDtypeStruct((_B, _D), jnp.float32),
    )(selfp, aggp, w1a, w1b, w2a, w2b)
    return h2


# speedup vs baseline: 1.0656x; 1.0463x over previous
import jax
import jax.numpy as jnp
import numpy as np
from jax import lax
from jax.experimental import pallas as pl
from jax.experimental.pallas import tpu as pltpu
from jax.experimental.pallas import tpu_sc as plsc

_D = 128
_K = 10
_B = 4096
_M = _B * (_K + 1)
_NW = 32
_PER_TILE = _M // _NW
_C = 16
_NCHUNK = _PER_TILE // _C
_NIDX = _C * (_K + 1)
_JROWS = 2
_JW = (_C * _K) // _JROWS


def _build_dmap():
    p = np.arange(_M)
    pn = p - _B
    r = np.where(p < _B, p, (pn % _K + 1) * _B + pn // _K)
    return r.reshape(_NW, _NCHUNK, _C).astype(np.int32)


_DMAP = _build_dmap()


def _sc_gather_mean(raw_hbm, nidx_hbm, sidx_hbm, dmap_hbm,
                    self_out, agg_out,
                    nidx_v, sidx_v, dmap_v, rows0, rows1, ag0, ag1,
                    sg0, sg1, sws0, sws1, swa0, swa1):
    cid = lax.axis_index("c")
    sid = lax.axis_index("s")
    wid = sid * 2 + cid

    rows = (rows0, rows1)
    aggs = (ag0, ag1)
    sem_g = (sg0, sg1)
    sem_ws = (sws0, sws1)
    sem_wa = (swa0, swa1)

    pltpu.sync_copy(nidx_hbm.at[wid], nidx_v)
    pltpu.sync_copy(sidx_hbm.at[wid], sidx_v)
    pltpu.sync_copy(dmap_hbm.at[wid], dmap_v)

    def g_issue(c, rows_ref, sem):
        for j in range(_JROWS):
            pltpu.async_copy(raw_hbm.at[nidx_v.at[c * _JROWS + j]],
                             rows_ref.at[pl.ds(j * _JW, _JW)], sem)
        pltpu.async_copy(raw_hbm.at[sidx_v.at[c]],
                         rows_ref.at[pl.ds(_C * _K, _C)], sem)

    def g_wait(c, rows_ref, sem):
        for j in range(_JROWS):
            pltpu.make_async_copy(raw_hbm.at[nidx_v.at[c * _JROWS + j]],
                                  rows_ref.at[pl.ds(j * _JW, _JW)], sem).wait()
        pltpu.make_async_copy(raw_hbm.at[sidx_v.at[c]],
                              rows_ref.at[pl.ds(_C * _K, _C)], sem).wait()

    def ws_issue(c, rows_ref, sem):
        pltpu.async_copy(rows_ref.at[pl.ds(_C * _K, _C)],
                         self_out.at[dmap_v.at[c]], sem)

    def wa_issue(c, agg_ref, sem):
        pltpu.async_copy(agg_ref, agg_out.at[dmap_v.at[c]], sem)

    def ws_wait(rows_ref, sem):
        pltpu.make_async_copy(rows_ref.at[pl.ds(_C * _K, _C)],
                              self_out.at[dmap_v.at[0]], sem).wait()

    def wa_wait(agg_ref, sem):
        pltpu.make_async_copy(agg_ref, agg_out.at[dmap_v.at[0]], sem).wait()

    def compute(rows_ref, agg_ref):
        def node_body(n, carry):
            base = n * _K
            for d in range(_D // 16):
                sl = pl.ds(d * 16, 16)
                r = [rows_ref[base + k, sl] for k in range(_K)]
                while len(r) > 1:
                    r = [r[i] + r[i + 1] for i in range(0, len(r) - 1, 2)] \
                        + ([r[-1]] if len(r) % 2 else [])
                agg_ref[n, sl] = r[0]
            return carry

        lax.fori_loop(0, _C, node_body, 0, unroll=False)

    def step(c, u, do_ws_wait, do_wa_wait, issue_next, when_pred=None):
        ru = u % 2
        ru1 = 1 - ru
        g_wait(c, rows[ru], sem_g[ru])
        ws_issue(c, rows[ru], sem_ws[ru])
        if issue_next:
            def _issue():
                if do_ws_wait:
                    ws_wait(rows[ru1], sem_ws[ru1])
                g_issue(c + 1, rows[ru1], sem_g[ru1])
            if when_pred is None:
                _issue()
            else:
                pl.when(when_pred)(_issue)
        if do_wa_wait:
            wa_wait(aggs[ru], sem_wa[ru])
        compute(rows[ru], aggs[ru])
        wa_issue(c, aggs[ru], sem_wa[ru])

    g_issue(0, rows0, sg0)
    step(0, 0, do_ws_wait=False, do_wa_wait=False, issue_next=True)
    step(1, 1, do_ws_wait=True, do_wa_wait=False, issue_next=True)

    def loop_body(t, carry):
        c0 = t * 2
        step(c0, 0, True, True, True)
        step(c0 + 1, 1, True, True, True, when_pred=t < (_NCHUNK // 2 - 1))
        return carry

    lax.fori_loop(1, _NCHUNK // 2, loop_body, 0)

    ws_wait(rows0, sws0)
    ws_wait(rows1, sws1)
    wa_wait(ag0, swa0)
    wa_wait(ag1, swa1)


def _tc_sage(self_ref, agg_ref, w1a, w1b, w2a, w2b, out_ref):
    h1s = jnp.maximum(
        jnp.dot(self_ref[0], w1a[...], preferred_element_type=jnp.float32)
        + jnp.dot(agg_ref[0], w1b[...], preferred_element_type=jnp.float32),
        0.0,
    )
    acc = jnp.zeros_like(h1s)
    for k in range(_K):
        hk = jnp.maximum(
            jnp.dot(self_ref[k + 1], w1a[...], preferred_element_type=jnp.float32)
            + jnp.dot(agg_ref[k + 1], w1b[...], preferred_element_type=jnp.float32),
            0.0,
        )
        acc = acc + hk
    out_ref[...] = jnp.maximum(
        jnp.dot(h1s, w2a[...], preferred_element_type=jnp.float32)
        + jnp.dot(acc, w2b[...], preferred_element_type=jnp.float32),
        0.0,
    )


def kernel(nodes_batch, raw_features, l1_nodes, neigh_l1, W1, W2):
    del nodes_batch
    nidx = neigh_l1.astype(jnp.int32).reshape(_NW, _NCHUNK * _JROWS, _JW)
    sidx = l1_nodes.astype(jnp.int32).reshape(_NW, _NCHUNK, _C)

    sc_fn = pl.kernel(
        _sc_gather_mean,
        out_type=(
            jax.ShapeDtypeStruct(((_K + 1) * _B, _D), jnp.float32),
            jax.ShapeDtypeStruct(((_K + 1) * _B, _D), jnp.float32),
        ),
        mesh=plsc.VectorSubcoreMesh(core_axis_name="c", subcore_axis_name="s"),
        scratch_types=[
            pltpu.VMEM((_NCHUNK * _JROWS, _JW), jnp.int32),
            pltpu.VMEM((_NCHUNK, _C), jnp.int32),
            pltpu.VMEM((_NCHUNK, _C), jnp.int32),
            pltpu.VMEM((_NIDX, _D), jnp.float32),
            pltpu.VMEM((_NIDX, _D), jnp.float32),
            pltpu.VMEM((_C, _D), jnp.float32),
            pltpu.VMEM((_C, _D), jnp.float32),
            pltpu.SemaphoreType.DMA,
            pltpu.SemaphoreType.DMA,
            pltpu.SemaphoreType.DMA,
            pltpu.SemaphoreType.DMA,
            pltpu.SemaphoreType.DMA,
            pltpu.SemaphoreType.DMA,
        ],
    )
    selfp, aggp = sc_fn(raw_features, nidx, sidx, jnp.asarray(_DMAP))

    w1a = W1[:, :_D].T
    w1b = W1[:, _D:].T * jnp.float32(1.0 / _K)
    w2a = W2[:, :_D].T
    w2b = W2[:, _D:].T * jnp.float32(1.0 / _K)
    selfp = selfp.reshape(_K + 1, _B, _D)
    aggp = aggp.reshape(_K + 1, _B, _D)

    S = 512
    grid = _B // S
    h2 = pl.pallas_call(
        _tc_sage,
        grid=(grid,),
        in_specs=[
            pl.BlockSpec((_K + 1, S, _D), lambda g: (0, g, 0)),
            pl.BlockSpec((_K + 1, S, _D), lambda g: (0, g, 0)),
            pl.BlockSpec((_D, _D), lambda g: (0, 0)),
            pl.BlockSpec((_D, _D), lambda g: (0, 0)),
            pl.BlockSpec((_D, _D), lambda g: (0, 0)),
            pl.BlockSpec((_D, _D), lambda g: (0, 0)),
        ],
        out_specs=pl.BlockSpec((S, _D), lambda g: (g, 0)),
        out_shape=jax.ShapeDtypeStruct((_B, _D), jnp.float32),
    )(selfp, aggp, w1a, w1b, w2a, w2b)
    return h2
